# baseline (speedup 1.0000x reference)
import functools

import jax
import jax.numpy as jnp
from jax.experimental import pallas as pl

MBLK = 64
NEG = -1e37


def _attn_kernel(q_ref, k_ref, v_ref, m_ref, o_ref, *, qt, tk):
    qi = pl.program_id(1)
    S = k_ref.shape[2]
    D = k_ref.shape[3]
    ng = qt // MBLK
    nbc = tk // MBLK

    q = q_ref[0, 0]

    G = (jax.lax.broadcasted_iota(jnp.int32, (ng, qt), 1) // MBLK ==
         jax.lax.broadcasted_iota(jnp.int32, (ng, qt), 0)).astype(jnp.float32)
    Ec = (jax.lax.broadcasted_iota(jnp.int32, (nbc, tk), 1) // MBLK ==
          jax.lax.broadcasted_iota(jnp.int32, (nbc, tk), 0)).astype(jnp.float32)
    R = (jax.lax.broadcasted_iota(jnp.int32, (qt, ng), 0) // MBLK ==
         jax.lax.broadcasted_iota(jnp.int32, (qt, ng), 1)).astype(jnp.float32)

    row_ids = qi * qt + jax.lax.broadcasted_iota(jnp.int32, (qt, tk), 0)
    col_iota = jax.lax.broadcasted_iota(jnp.int32, (qt, tk), 1)

    def step(j, carry, causal):
        m_i, l_i, acc = carry
        k = k_ref[0, 0, pl.ds(j * tk, tk), :]
        v = v_ref[0, 0, pl.ds(j * tk, tk), :]
        s = jax.lax.dot_general(
            q, k, (((1,), (1,)), ((), ())),
            preferred_element_type=jnp.float32)
        mc = m_ref[0, :, pl.ds(j * tk, tk)].astype(jnp.float32)
        colsum = jax.lax.dot_general(
            G, mc, (((1,), (0,)), ((), ())),
            preferred_element_type=jnp.float32)
        bsum = jax.lax.dot_general(
            colsum, Ec, (((1,), (1,)), ((), ())),
            preferred_element_type=jnp.float32)
        pre = jnp.where(bsum > 0, 0.0, NEG)
        biasrow = jax.lax.dot_general(
            pre, Ec, (((1,), (0,)), ((), ())),
            preferred_element_type=jnp.float32)
        bias = jax.lax.dot_general(
            R, biasrow, (((1,), (0,)), ((), ())),
            preferred_element_type=jnp.float32)
        s = s + bias
        if causal:
            s = jnp.where(j * tk + col_iota <= row_ids, s, NEG)
        m_new = jnp.maximum(m_i, jnp.max(s, axis=1, keepdims=True))
        p = jnp.exp(s - m_new)
        alpha = jnp.exp(m_i - m_new)
        l_new = l_i * alpha + jnp.sum(p, axis=1, keepdims=True)
        acc_new = acc * alpha + jax.lax.dot_general(
            p, v, (((1,), (0,)), ((), ())),
            preferred_element_type=jnp.float32)
        return m_new, l_new, acc_new

    n_chunks = qi * qt // tk + 1
    m0 = jnp.full((qt, 1), NEG, jnp.float32)
    l0 = jnp.zeros((qt, 1), jnp.float32)
    acc0 = jnp.zeros((qt, D), jnp.float32)
    carry = jax.lax.fori_loop(
        0, n_chunks - 1, lambda j, c: step(j, c, causal=False), (m0, l0, acc0))
    m_f, l_f, acc_f = step(n_chunks - 1, carry, causal=True)

    out = jnp.where(m_f > NEG * 0.1, acc_f / l_f, 0.0)
    o_ref[0, 0] = out


@jax.jit
def kernel(query, key, value, mask):
    B, H, S, D = query.shape
    qt = min(256, S)
    tk = min(512, S)
    grid = (H, S // qt)
    out = pl.pallas_call(
        functools.partial(_attn_kernel, qt=qt, tk=tk),
        grid=grid,
        in_specs=[
            pl.BlockSpec((1, 1, qt, D), lambda h, i: (0, h, i, 0)),
            pl.BlockSpec((1, 1, S, D), lambda h, i: (0, h, 0, 0)),
            pl.BlockSpec((1, 1, S, D), lambda h, i: (0, h, 0, 0)),
            pl.BlockSpec((1, qt, S), lambda h, i: (h, i, 0)),
        ],
        out_specs=pl.BlockSpec((1, 1, qt, D), lambda h, i: (0, h, i, 0)),
        out_shape=jax.ShapeDtypeStruct((B, H, S, D), jnp.float32),
    )(query, key, value, mask)
    return out
